# Initial kernel scaffold; baseline (speedup 1.0000x reference)
import jax
import jax.numpy as jnp
from jax.experimental import pallas as pl


def kernel(x, edge_index, batch, edge_attr, Wemb, bemb, W1, b1, W2, W3, b3, Wf1, bf1, Wf2, bf2):
    raise NotImplementedError("write your pallas kernel here")


# speedup vs baseline: 5.2335x; 5.2335x over previous
import functools

import jax
import jax.numpy as jnp
from jax import lax
from jax.experimental import pallas as pl
from jax.experimental.pallas import tpu as pltpu
from jax.experimental.pallas import tpu_sc as plsc


def _make_edge_agg(n_pad, n_edges, width, n_super, cps, chunk):
    info = plsc.get_sparse_core_info()
    n_cores, n_sub, L = info.num_cores, info.num_subcores, info.num_lanes
    nw = n_cores * n_sub
    ep = n_edges // nw
    assert ep == n_super * cps * chunk
    rps = n_pad // n_sub
    zr = 64
    assert rps % zr == 0 and rps % 8 == 0
    sedge = cps * chunk

    mesh = plsc.VectorSubcoreMesh(core_axis_name="c", subcore_axis_name="s")

    @functools.partial(
        pl.kernel,
        out_type=jax.ShapeDtypeStruct((n_cores, n_pad, width), jnp.float32),
        mesh=mesh,
        compiler_params=pltpu.CompilerParams(needs_layout_passes=False),
        scratch_types=[
            pltpu.VMEM((cps, chunk), jnp.int32),
            pltpu.VMEM((cps, chunk), jnp.int32),
            pltpu.VMEM((sedge,), jnp.float32),
            pltpu.VMEM((chunk, width), jnp.float32),
            pltpu.VMEM((zr, width), jnp.float32),
            pltpu.VMEM_SHARED((n_pad, width), jnp.float32),
            pltpu.SemaphoreType.DMA,
        ],
    )
    def edge_agg(table_hbm, src_hbm, dst_hbm, w_hbm, out_hbm,
                 srcv, dstv, wv, rows, zbuf, acc, sem):
        c = lax.axis_index("c")
        s = lax.axis_index("s")
        wid = s * n_cores + c

        zeros = jnp.zeros((L,), jnp.float32)

        def zrow(r, _):
            for j in range(width // L):
                zbuf[r, pl.ds(j * L, L)] = zeros
            return 0

        lax.fori_loop(0, zr, zrow, 0)
        for t in range(rps // zr):
            pltpu.sync_copy(zbuf, acc.at[pl.ds(s * rps + t * zr, zr)])
        plsc.subcore_barrier()

        def do_super(si, _):
            pltpu.sync_copy(src_hbm.at[wid, si], srcv)
            pltpu.sync_copy(dst_hbm.at[wid, si], dstv)
            pltpu.sync_copy(w_hbm.at[wid, si], wv)

            def do_chunk(i, _):
                pltpu.async_copy(table_hbm.at[srcv.at[i]], rows, sem).wait()

                def scale(r, _):
                    wb = plsc.load_gather(
                        wv, [jnp.full((L,), i * chunk + r, jnp.int32)])
                    for j in range(width // L):
                        rows[r, pl.ds(j * L, L)] = rows[r, pl.ds(j * L, L)] * wb
                    return 0

                lax.fori_loop(0, chunk, scale, 0)
                pltpu.sync_copy(rows, acc.at[dstv.at[i]], add=True)
                return 0

            lax.fori_loop(0, cps, do_chunk, 0)
            return 0

        lax.fori_loop(0, n_super, do_super, 0)
        plsc.subcore_barrier()

        pltpu.sync_copy(acc.at[pl.ds(s * rps, rps)],
                        out_hbm.at[c, pl.ds(s * rps, rps)])

    return edge_agg


def _dot(a, b):
    return jnp.dot(a, b, preferred_element_type=jnp.float32)


def _tc_embed(x, wemb, bemb, w1, b1, *, br):
    n, d = x.shape
    h = wemb.shape[1]

    def body(x_ref, we, be, w1r, b1r, h_ref, a_ref):
        hh = _dot(x_ref[...], we[...]) + be[...]
        h_ref[...] = hh
        a_ref[...] = _dot(hh, w1r[...]) + b1r[...]

    return pl.pallas_call(
        body,
        grid=(n // br,),
        in_specs=[
            pl.BlockSpec((br, d), lambda i: (i, 0)),
            pl.BlockSpec((d, h), lambda i: (0, 0)),
            pl.BlockSpec((1, h), lambda i: (0, 0)),
            pl.BlockSpec((h, h), lambda i: (0, 0)),
            pl.BlockSpec((1, h), lambda i: (0, 0)),
        ],
        out_specs=[pl.BlockSpec((br, h), lambda i: (i, 0)),
                   pl.BlockSpec((br, h), lambda i: (i, 0))],
        out_shape=[jax.ShapeDtypeStruct((n, h), jnp.float32),
                   jax.ShapeDtypeStruct((n, h), jnp.float32)],
    )(x, wemb, bemb, w1, b1)


def _tc_update(p0, p1, hcur, d0, d1, w2, w3, b3, w1n, b1n, *, br):
    n, h = hcur.shape
    dw = d0.shape[1]

    def body(p0r, p1r, hr, d0r, d1r, w2r, w3r, b3r, w1r, b1r, hn_ref, an_ref):
        deg = d0r[...][:, 0:1] + d1r[...][:, 0:1]
        hh = hr[...]
        hnew = jax.nn.relu(p0r[...] + p1r[...] - _dot(hh, w2r[...]) * deg
                           + _dot(hh, w3r[...]) + b3r[...])
        hn_ref[...] = hnew
        an_ref[...] = _dot(hnew, w1r[...]) + b1r[...]

    rowspec = pl.BlockSpec((br, h), lambda i: (i, 0))
    wspec = pl.BlockSpec((h, h), lambda i: (0, 0))
    bspec = pl.BlockSpec((1, h), lambda i: (0, 0))
    dspec = pl.BlockSpec((br, dw), lambda i: (i, 0))
    return pl.pallas_call(
        body,
        grid=(n // br,),
        in_specs=[rowspec, rowspec, rowspec, dspec, dspec,
                  wspec, wspec, bspec, wspec, bspec],
        out_specs=[rowspec, rowspec],
        out_shape=[jax.ShapeDtypeStruct((n, h), jnp.float32),
                   jax.ShapeDtypeStruct((n, h), jnp.float32)],
    )(p0, p1, hcur, d0, d1, w2, w3, b3, w1n, b1n)


def _tc_final(p0, p1, hcur, d0, d1, w2, w3, b3, batch2d, wf1, bf1, wf2p, bf2p,
              *, br, ng):
    n, h = hcur.shape
    dw = d0.shape[1]
    h2 = wf1.shape[1]
    op = wf2p.shape[1]
    nsteps = n // br

    def body(p0r, p1r, hr, d0r, d1r, w2r, w3r, b3r, br_ref, wf1r, bf1r,
             wf2r, bf2r, out_ref, sums, counts):
        i = pl.program_id(0)

        @pl.when(i == 0)
        def _init():
            sums[...] = jnp.zeros_like(sums)
            counts[...] = jnp.zeros_like(counts)

        deg = d0r[...][:, 0:1] + d1r[...][:, 0:1]
        hh = hr[...]
        hnew = jax.nn.relu(p0r[...] + p1r[...] - _dot(hh, w2r[...]) * deg
                           + _dot(hh, w3r[...]) + b3r[...])
        onehot = (br_ref[...] == lax.broadcasted_iota(jnp.int32, (br, ng), 1)
                  ).astype(jnp.float32)
        sums[...] += lax.dot_general(onehot, hnew, (((0,), (0,)), ((), ())),
                                     preferred_element_type=jnp.float32)
        counts[...] += lax.dot_general(onehot, jnp.ones((br, h), jnp.float32),
                                       (((0,), (0,)), ((), ())),
                                       preferred_element_type=jnp.float32)

        @pl.when(i == nsteps - 1)
        def _fin():
            gx = sums[...] / jnp.maximum(counts[...], 1.0)
            hid = jax.nn.relu(_dot(gx, wf1r[...]) + bf1r[...])
            out_ref[...] = _dot(hid, wf2r[...]) + bf2r[...]

    rowspec = pl.BlockSpec((br, h), lambda i: (i, 0))
    wspec = pl.BlockSpec((h, h), lambda i: (0, 0))
    bspec = pl.BlockSpec((1, h), lambda i: (0, 0))
    dspec = pl.BlockSpec((br, dw), lambda i: (i, 0))
    return pl.pallas_call(
        body,
        grid=(nsteps,),
        in_specs=[rowspec, rowspec, rowspec, dspec, dspec, wspec, wspec, bspec,
                  pl.BlockSpec((br, 1), lambda i: (i, 0)),
                  pl.BlockSpec((h, h2), lambda i: (0, 0)),
                  pl.BlockSpec((1, h2), lambda i: (0, 0)),
                  pl.BlockSpec((h2, op), lambda i: (0, 0)),
                  pl.BlockSpec((1, op), lambda i: (0, 0))],
        out_specs=pl.BlockSpec((ng, op), lambda i: (0, 0)),
        out_shape=jax.ShapeDtypeStruct((ng, op), jnp.float32),
        scratch_shapes=[pltpu.VMEM((ng, h), jnp.float32),
                        pltpu.VMEM((ng, h), jnp.float32)],
    )(p0, p1, hcur, d0, d1, w2, w3, b3, batch2d, wf1, bf1, wf2p, bf2p)


def kernel(x, edge_index, batch, edge_attr, Wemb, bemb, W1, b1, W2, W3, b3,
           Wf1, bf1, Wf2, bf2):
    n, d = x.shape
    e = edge_index.shape[1]
    h = Wemb.shape[1]
    nl = W1.shape[0]
    ng = 128
    nc = Wf2.shape[1]

    nw = 32
    chunk = 80
    cps = 25
    n_super = e // nw // (cps * chunk)
    br = 2000
    n_pad = 10240

    src3 = edge_index[0].reshape(nw, n_super, cps, chunk)
    dst3 = edge_index[1].reshape(nw, n_super, cps, chunk)
    w3d = edge_attr.reshape(nw, n_super, cps * chunk)
    batch2d = batch.reshape(n, 1)

    bemb2 = bemb.reshape(1, h)
    b1r = b1.reshape(nl, 1, h)
    b3r = b3.reshape(nl, 1, h)
    bf1r = bf1.reshape(1, 2 * h)
    wf2p = jnp.pad(Wf2, ((0, 0), (0, h - nc)))
    bf2p = jnp.pad(bf2, (0, h - nc)).reshape(1, h)

    agg_full = _make_edge_agg(n_pad, e, h, n_super, cps, chunk)
    ones128 = jnp.ones((n, h), jnp.float32)

    degp = agg_full(ones128, src3, dst3, w3d)
    d0, d1 = degp[0, :n, :16], degp[1, :n, :16]

    hh, a = _tc_embed(x, Wemb, bemb2, W1[0], b1r[0], br=br)
    for l in range(nl):
        p = agg_full(a, src3, dst3, w3d)
        if l < nl - 1:
            hh, a = _tc_update(p[0, :n], p[1, :n], hh, d0, d1, W2[l], W3[l],
                               b3r[l], W1[l + 1], b1r[l + 1], br=br)
        else:
            pred = _tc_final(p[0, :n], p[1, :n], hh, d0, d1, W2[l], W3[l],
                             b3r[l], batch2d, Wf1, bf1r, wf2p, bf2p, br=br,
                             ng=ng)
    return pred[:, :nc]


# speedup vs baseline: 5.6917x; 1.0875x over previous
import functools

import jax
import jax.numpy as jnp
from jax import lax
from jax.experimental import pallas as pl
from jax.experimental.pallas import tpu as pltpu
from jax.experimental.pallas import tpu_sc as plsc


def _make_edge_agg(n_pad, n_edges, width, n_super, cps, chunk):
    info = plsc.get_sparse_core_info()
    n_cores, n_sub, L = info.num_cores, info.num_subcores, info.num_lanes
    nw = n_cores * n_sub
    ep = n_edges // nw
    assert ep == n_super * cps * chunk
    rps = n_pad // n_sub
    zr = 64
    assert rps % zr == 0 and rps % 8 == 0
    sedge = cps * chunk

    mesh = plsc.VectorSubcoreMesh(core_axis_name="c", subcore_axis_name="s")

    @functools.partial(
        pl.kernel,
        out_type=jax.ShapeDtypeStruct((n_cores, n_pad, width), jnp.float32),
        mesh=mesh,
        compiler_params=pltpu.CompilerParams(needs_layout_passes=False),
        scratch_types=[
            pltpu.VMEM((cps, chunk), jnp.int32),
            pltpu.VMEM((cps, chunk), jnp.int32),
            pltpu.VMEM((sedge,), jnp.float32),
            pltpu.VMEM((chunk, width), jnp.float32),
            pltpu.VMEM((zr, width), jnp.float32),
            pltpu.VMEM_SHARED((n_pad, width), jnp.float32),
            pltpu.SemaphoreType.DMA,
        ],
    )
    def edge_agg(table_hbm, src_hbm, dst_hbm, w_hbm, out_hbm,
                 srcv, dstv, wv, rows, zbuf, acc, sem):
        c = lax.axis_index("c")
        s = lax.axis_index("s")
        wid = s * n_cores + c

        zeros = jnp.zeros((L,), jnp.float32)

        def zrow(r, _):
            for j in range(width // L):
                zbuf[r, pl.ds(j * L, L)] = zeros
            return 0

        lax.fori_loop(0, zr, zrow, 0)
        for t in range(rps // zr):
            pltpu.sync_copy(zbuf, acc.at[pl.ds(s * rps + t * zr, zr)])
        plsc.subcore_barrier()

        def do_super(si, _):
            pltpu.sync_copy(src_hbm.at[wid, si], srcv)
            pltpu.sync_copy(dst_hbm.at[wid, si], dstv)
            pltpu.sync_copy(w_hbm.at[wid, si], wv)

            def do_chunk(i, _):
                pltpu.async_copy(table_hbm.at[srcv.at[i]], rows, sem).wait()

                def scale(r, _):
                    wb = plsc.load_gather(
                        wv, [jnp.full((L,), i * chunk + r, jnp.int32)])
                    for j in range(width // L):
                        rows[r, pl.ds(j * L, L)] = rows[r, pl.ds(j * L, L)] * wb
                    return 0

                lax.fori_loop(0, chunk, scale, 0)
                pltpu.sync_copy(rows, acc.at[dstv.at[i]], add=True)
                return 0

            lax.fori_loop(0, cps, do_chunk, 0)
            return 0

        lax.fori_loop(0, n_super, do_super, 0)
        plsc.subcore_barrier()

        pltpu.sync_copy(acc.at[pl.ds(s * rps, rps)],
                        out_hbm.at[c, pl.ds(s * rps, rps)])

    return edge_agg


def _dot(a, b):
    return jnp.dot(a, b, preferred_element_type=jnp.float32)


def _tc_embed(x, wemb, bemb, w1, b1, *, br):
    n, d = x.shape
    h = wemb.shape[1]

    def body(x_ref, we, be, w1r, b1r, h_ref, a_ref):
        hh = _dot(x_ref[...], we[...]) + be[...]
        h_ref[...] = hh
        a_ref[...] = _dot(hh, w1r[...]) + b1r[...]

    return pl.pallas_call(
        body,
        grid=(n // br,),
        in_specs=[
            pl.BlockSpec((br, d), lambda i: (i, 0)),
            pl.BlockSpec((d, h), lambda i: (0, 0)),
            pl.BlockSpec((1, h), lambda i: (0, 0)),
            pl.BlockSpec((h, h), lambda i: (0, 0)),
            pl.BlockSpec((1, h), lambda i: (0, 0)),
        ],
        out_specs=[pl.BlockSpec((br, h), lambda i: (i, 0)),
                   pl.BlockSpec((br, h), lambda i: (i, 0))],
        out_shape=[jax.ShapeDtypeStruct((n, h), jnp.float32),
                   jax.ShapeDtypeStruct((n, h), jnp.float32)],
    )(x, wemb, bemb, w1, b1)


def _tc_update(p0, p1, hcur, d0, d1, w2, w3, b3, w1n, b1n, *, br):
    n, h = hcur.shape
    dw = d0.shape[1]

    def body(p0r, p1r, hr, d0r, d1r, w2r, w3r, b3r, w1r, b1r, hn_ref, an_ref):
        deg = d0r[...][:, 0:1] + d1r[...][:, 0:1]
        hh = hr[...]
        hnew = jax.nn.relu(p0r[...] + p1r[...] - _dot(hh, w2r[...]) * deg
                           + _dot(hh, w3r[...]) + b3r[...])
        hn_ref[...] = hnew
        an_ref[...] = _dot(hnew, w1r[...]) + b1r[...]

    rowspec = pl.BlockSpec((br, h), lambda i: (i, 0))
    wspec = pl.BlockSpec((h, h), lambda i: (0, 0))
    bspec = pl.BlockSpec((1, h), lambda i: (0, 0))
    dspec = pl.BlockSpec((br, dw), lambda i: (i, 0))
    return pl.pallas_call(
        body,
        grid=(n // br,),
        in_specs=[rowspec, rowspec, rowspec, dspec, dspec,
                  wspec, wspec, bspec, wspec, bspec],
        out_specs=[rowspec, rowspec],
        out_shape=[jax.ShapeDtypeStruct((n, h), jnp.float32),
                   jax.ShapeDtypeStruct((n, h), jnp.float32)],
    )(p0, p1, hcur, d0, d1, w2, w3, b3, w1n, b1n)


def _tc_final(p0, p1, hcur, d0, d1, w2, w3, b3, batch2d, wf1, bf1, wf2p, bf2p,
              *, br, ng):
    n, h = hcur.shape
    dw = d0.shape[1]
    h2 = wf1.shape[1]
    op = wf2p.shape[1]
    nsteps = n // br

    def body(p0r, p1r, hr, d0r, d1r, w2r, w3r, b3r, br_ref, wf1r, bf1r,
             wf2r, bf2r, out_ref, sums, counts):
        i = pl.program_id(0)

        @pl.when(i == 0)
        def _init():
            sums[...] = jnp.zeros_like(sums)
            counts[...] = jnp.zeros_like(counts)

        deg = d0r[...][:, 0:1] + d1r[...][:, 0:1]
        hh = hr[...]
        hnew = jax.nn.relu(p0r[...] + p1r[...] - _dot(hh, w2r[...]) * deg
                           + _dot(hh, w3r[...]) + b3r[...])
        onehot = (br_ref[...] == lax.broadcasted_iota(jnp.int32, (br, ng), 1)
                  ).astype(jnp.float32)
        sums[...] += lax.dot_general(onehot, hnew, (((0,), (0,)), ((), ())),
                                     preferred_element_type=jnp.float32)
        counts[...] += lax.dot_general(onehot, jnp.ones((br, h), jnp.float32),
                                       (((0,), (0,)), ((), ())),
                                       preferred_element_type=jnp.float32)

        @pl.when(i == nsteps - 1)
        def _fin():
            gx = sums[...] / jnp.maximum(counts[...], 1.0)
            hid = jax.nn.relu(_dot(gx, wf1r[...]) + bf1r[...])
            out_ref[...] = _dot(hid, wf2r[...]) + bf2r[...]

    rowspec = pl.BlockSpec((br, h), lambda i: (i, 0))
    wspec = pl.BlockSpec((h, h), lambda i: (0, 0))
    bspec = pl.BlockSpec((1, h), lambda i: (0, 0))
    dspec = pl.BlockSpec((br, dw), lambda i: (i, 0))
    return pl.pallas_call(
        body,
        grid=(nsteps,),
        in_specs=[rowspec, rowspec, rowspec, dspec, dspec, wspec, wspec, bspec,
                  pl.BlockSpec((br, 1), lambda i: (i, 0)),
                  pl.BlockSpec((h, h2), lambda i: (0, 0)),
                  pl.BlockSpec((1, h2), lambda i: (0, 0)),
                  pl.BlockSpec((h2, op), lambda i: (0, 0)),
                  pl.BlockSpec((1, op), lambda i: (0, 0))],
        out_specs=pl.BlockSpec((ng, op), lambda i: (0, 0)),
        out_shape=jax.ShapeDtypeStruct((ng, op), jnp.float32),
        scratch_shapes=[pltpu.VMEM((ng, h), jnp.float32),
                        pltpu.VMEM((ng, h), jnp.float32)],
    )(p0, p1, hcur, d0, d1, w2, w3, b3, batch2d, wf1, bf1, wf2p, bf2p)


def kernel(x, edge_index, batch, edge_attr, Wemb, bemb, W1, b1, W2, W3, b3,
           Wf1, bf1, Wf2, bf2):
    n, d = x.shape
    e = edge_index.shape[1]
    h = Wemb.shape[1]
    nl = W1.shape[0]
    ng = 128
    nc = Wf2.shape[1]

    nw = 32
    chunk = 128
    cps = 16
    n_super = 5
    br = 2000
    n_pad = 10240

    ep = n_super * cps * chunk
    e_pad = nw * ep
    npd = e_pad - e
    fill = jnp.arange(npd, dtype=jnp.int32)
    src_p = jnp.concatenate([edge_index[0], fill % n])
    dst_p = jnp.concatenate([edge_index[1], fill % n_pad])
    w_p = jnp.pad(edge_attr, (0, npd))
    src3 = src_p.reshape(nw, n_super, cps, chunk)
    dst3 = dst_p.reshape(nw, n_super, cps, chunk)
    w3d = w_p.reshape(nw, n_super, cps * chunk)
    batch2d = batch.reshape(n, 1)

    bemb2 = bemb.reshape(1, h)
    b1r = b1.reshape(nl, 1, h)
    b3r = b3.reshape(nl, 1, h)
    bf1r = bf1.reshape(1, 2 * h)
    wf2p = jnp.pad(Wf2, ((0, 0), (0, h - nc)))
    bf2p = jnp.pad(bf2, (0, h - nc)).reshape(1, h)

    agg_full = _make_edge_agg(n_pad, e_pad, h, n_super, cps, chunk)
    ones128 = jnp.ones((n, h), jnp.float32)

    degp = agg_full(ones128, src3, dst3, w3d)
    d0, d1 = degp[0, :n, :16], degp[1, :n, :16]

    hh, a = _tc_embed(x, Wemb, bemb2, W1[0], b1r[0], br=br)
    for l in range(nl):
        p = agg_full(a, src3, dst3, w3d)
        if l < nl - 1:
            hh, a = _tc_update(p[0, :n], p[1, :n], hh, d0, d1, W2[l], W3[l],
                               b3r[l], W1[l + 1], b1r[l + 1], br=br)
        else:
            pred = _tc_final(p[0, :n], p[1, :n], hh, d0, d1, W2[l], W3[l],
                             b3r[l], batch2d, Wf1, bf1r, wf2p, bf2p, br=br,
                             ng=ng)
    return pred[:, :nc]


# speedup vs baseline: 8.1851x; 1.4381x over previous
import functools

import jax
import jax.numpy as jnp
from jax import lax
from jax.experimental import pallas as pl
from jax.experimental.pallas import tpu as pltpu
from jax.experimental.pallas import tpu_sc as plsc


def _make_edge_agg(n_pad, n_edges, width, n_super, cps, chunk):
    info = plsc.get_sparse_core_info()
    n_cores, n_sub, L = info.num_cores, info.num_subcores, info.num_lanes
    nw = n_cores * n_sub
    ep = n_edges // nw
    assert ep == n_super * cps * chunk
    rps = n_pad // n_sub
    zr = 64
    assert rps % zr == 0 and rps % 8 == 0
    sedge = cps * chunk

    mesh = plsc.VectorSubcoreMesh(core_axis_name="c", subcore_axis_name="s")

    @functools.partial(
        pl.kernel,
        out_type=jax.ShapeDtypeStruct((n_cores, n_pad, width), jnp.float32),
        mesh=mesh,
        compiler_params=pltpu.CompilerParams(needs_layout_passes=False),
        scratch_types=[
            pltpu.VMEM((cps, chunk), jnp.int32),
            pltpu.VMEM((cps, chunk), jnp.int32),
            pltpu.VMEM((sedge,), jnp.float32),
            pltpu.VMEM((chunk, width), jnp.float32),
            pltpu.VMEM((chunk, width), jnp.float32),
            pltpu.VMEM((zr, width), jnp.float32),
            pltpu.VMEM_SHARED((n_pad, width), jnp.float32),
            pltpu.SemaphoreType.DMA,
            pltpu.SemaphoreType.DMA,
            pltpu.SemaphoreType.DMA,
            pltpu.SemaphoreType.DMA,
        ],
    )
    def edge_agg(table_hbm, src_hbm, dst_hbm, w_hbm, out_hbm,
                 srcv, dstv, wv, rows_a, rows_b, zbuf, acc,
                 gsem_a, gsem_b, ssem_a, ssem_b):
        c = lax.axis_index("c")
        s = lax.axis_index("s")
        wid = s * n_cores + c

        rows = {0: rows_a, 1: rows_b}
        gsem = {0: gsem_a, 1: gsem_b}
        ssem = {0: ssem_a, 1: ssem_b}

        def g_start(b, k):
            pltpu.async_copy(table_hbm.at[srcv.at[k]], rows[b], gsem[b])

        def g_wait(b):
            pltpu.make_async_copy(
                table_hbm.at[srcv.at[0]], rows[b], gsem[b]).wait()

        def s_start(b, k):
            pltpu.async_copy(rows[b], acc.at[dstv.at[k]], ssem[b], add=True)

        def s_wait(b):
            pltpu.make_async_copy(rows[b], acc.at[dstv.at[0]], ssem[b]).wait()

        def scale(b, k):
            buf = rows[b]

            def sbody(r, _):
                wb = plsc.load_gather(
                    wv, [jnp.full((L,), k * chunk + r, jnp.int32)])
                for j in range(width // L):
                    buf[r, pl.ds(j * L, L)] = buf[r, pl.ds(j * L, L)] * wb
                return 0

            lax.fori_loop(0, chunk, sbody, 0)

        zeros = jnp.zeros((L,), jnp.float32)

        def zrow(r, _):
            for j in range(width // L):
                zbuf[r, pl.ds(j * L, L)] = zeros
            return 0

        lax.fori_loop(0, zr, zrow, 0)
        for t in range(rps // zr):
            pltpu.sync_copy(zbuf, acc.at[pl.ds(s * rps + t * zr, zr)])
        plsc.subcore_barrier()

        def do_super(si, _):
            pltpu.sync_copy(src_hbm.at[wid, si], srcv)
            pltpu.sync_copy(dst_hbm.at[wid, si], dstv)
            pltpu.sync_copy(w_hbm.at[wid, si], wv)

            g_start(0, 0)
            g_wait(0)
            g_start(1, 1)
            scale(0, 0)
            s_start(0, 0)

            def pair(j, _):
                s_wait(0)
                g_start(0, 2 * j)
                g_wait(1)
                scale(1, 2 * j - 1)
                s_start(1, 2 * j - 1)
                g_wait(0)
                scale(0, 2 * j)
                s_wait(1)
                g_start(1, 2 * j + 1)
                s_start(0, 2 * j)
                return 0

            lax.fori_loop(1, cps // 2, pair, 0)

            g_wait(1)
            scale(1, cps - 1)
            s_wait(0)
            s_start(1, cps - 1)
            s_wait(1)
            return 0

        lax.fori_loop(0, n_super, do_super, 0)
        plsc.subcore_barrier()

        pltpu.sync_copy(acc.at[pl.ds(s * rps, rps)],
                        out_hbm.at[c, pl.ds(s * rps, rps)])

    return edge_agg


def _dot(a, b):
    return jnp.dot(a, b, preferred_element_type=jnp.float32)


def _tc_embed(x, wemb, bemb, w1, b1, *, br):
    n, d = x.shape
    h = wemb.shape[1]

    def body(x_ref, we, be, w1r, b1r, h_ref, a_ref):
        hh = _dot(x_ref[...], we[...]) + be[...]
        h_ref[...] = hh
        a_ref[...] = _dot(hh, w1r[...]) + b1r[...]

    return pl.pallas_call(
        body,
        grid=(n // br,),
        in_specs=[
            pl.BlockSpec((br, d), lambda i: (i, 0)),
            pl.BlockSpec((d, h), lambda i: (0, 0)),
            pl.BlockSpec((1, h), lambda i: (0, 0)),
            pl.BlockSpec((h, h), lambda i: (0, 0)),
            pl.BlockSpec((1, h), lambda i: (0, 0)),
        ],
        out_specs=[pl.BlockSpec((br, h), lambda i: (i, 0)),
                   pl.BlockSpec((br, h), lambda i: (i, 0))],
        out_shape=[jax.ShapeDtypeStruct((n, h), jnp.float32),
                   jax.ShapeDtypeStruct((n, h), jnp.float32)],
    )(x, wemb, bemb, w1, b1)


def _tc_update(p0, p1, hcur, d0, d1, w2, w3, b3, w1n, b1n, *, br):
    n, h = hcur.shape
    dw = d0.shape[1]

    def body(p0r, p1r, hr, d0r, d1r, w2r, w3r, b3r, w1r, b1r, hn_ref, an_ref):
        deg = d0r[...][:, 0:1] + d1r[...][:, 0:1]
        hh = hr[...]
        hnew = jax.nn.relu(p0r[...] + p1r[...] - _dot(hh, w2r[...]) * deg
                           + _dot(hh, w3r[...]) + b3r[...])
        hn_ref[...] = hnew
        an_ref[...] = _dot(hnew, w1r[...]) + b1r[...]

    rowspec = pl.BlockSpec((br, h), lambda i: (i, 0))
    wspec = pl.BlockSpec((h, h), lambda i: (0, 0))
    bspec = pl.BlockSpec((1, h), lambda i: (0, 0))
    dspec = pl.BlockSpec((br, dw), lambda i: (i, 0))
    return pl.pallas_call(
        body,
        grid=(n // br,),
        in_specs=[rowspec, rowspec, rowspec, dspec, dspec,
                  wspec, wspec, bspec, wspec, bspec],
        out_specs=[rowspec, rowspec],
        out_shape=[jax.ShapeDtypeStruct((n, h), jnp.float32),
                   jax.ShapeDtypeStruct((n, h), jnp.float32)],
    )(p0, p1, hcur, d0, d1, w2, w3, b3, w1n, b1n)


def _tc_final(p0, p1, hcur, d0, d1, w2, w3, b3, batch2d, wf1, bf1, wf2p, bf2p,
              *, br, ng):
    n, h = hcur.shape
    dw = d0.shape[1]
    h2 = wf1.shape[1]
    op = wf2p.shape[1]
    nsteps = n // br

    def body(p0r, p1r, hr, d0r, d1r, w2r, w3r, b3r, br_ref, wf1r, bf1r,
             wf2r, bf2r, out_ref, sums, counts):
        i = pl.program_id(0)

        @pl.when(i == 0)
        def _init():
            sums[...] = jnp.zeros_like(sums)
            counts[...] = jnp.zeros_like(counts)

        deg = d0r[...][:, 0:1] + d1r[...][:, 0:1]
        hh = hr[...]
        hnew = jax.nn.relu(p0r[...] + p1r[...] - _dot(hh, w2r[...]) * deg
                           + _dot(hh, w3r[...]) + b3r[...])
        onehot = (br_ref[...] == lax.broadcasted_iota(jnp.int32, (br, ng), 1)
                  ).astype(jnp.float32)
        sums[...] += lax.dot_general(onehot, hnew, (((0,), (0,)), ((), ())),
                                     preferred_element_type=jnp.float32)
        counts[...] += lax.dot_general(onehot, jnp.ones((br, h), jnp.float32),
                                       (((0,), (0,)), ((), ())),
                                       preferred_element_type=jnp.float32)

        @pl.when(i == nsteps - 1)
        def _fin():
            gx = sums[...] / jnp.maximum(counts[...], 1.0)
            hid = jax.nn.relu(_dot(gx, wf1r[...]) + bf1r[...])
            out_ref[...] = _dot(hid, wf2r[...]) + bf2r[...]

    rowspec = pl.BlockSpec((br, h), lambda i: (i, 0))
    wspec = pl.BlockSpec((h, h), lambda i: (0, 0))
    bspec = pl.BlockSpec((1, h), lambda i: (0, 0))
    dspec = pl.BlockSpec((br, dw), lambda i: (i, 0))
    return pl.pallas_call(
        body,
        grid=(nsteps,),
        in_specs=[rowspec, rowspec, rowspec, dspec, dspec, wspec, wspec, bspec,
                  pl.BlockSpec((br, 1), lambda i: (i, 0)),
                  pl.BlockSpec((h, h2), lambda i: (0, 0)),
                  pl.BlockSpec((1, h2), lambda i: (0, 0)),
                  pl.BlockSpec((h2, op), lambda i: (0, 0)),
                  pl.BlockSpec((1, op), lambda i: (0, 0))],
        out_specs=pl.BlockSpec((ng, op), lambda i: (0, 0)),
        out_shape=jax.ShapeDtypeStruct((ng, op), jnp.float32),
        scratch_shapes=[pltpu.VMEM((ng, h), jnp.float32),
                        pltpu.VMEM((ng, h), jnp.float32)],
    )(p0, p1, hcur, d0, d1, w2, w3, b3, batch2d, wf1, bf1, wf2p, bf2p)


def kernel(x, edge_index, batch, edge_attr, Wemb, bemb, W1, b1, W2, W3, b3,
           Wf1, bf1, Wf2, bf2):
    n, d = x.shape
    e = edge_index.shape[1]
    h = Wemb.shape[1]
    nl = W1.shape[0]
    ng = 128
    nc = Wf2.shape[1]

    nw = 32
    chunk = 128
    cps = 16
    n_super = 5
    br = 2000
    n_pad = 10240

    ep = n_super * cps * chunk
    e_pad = nw * ep
    npd = e_pad - e
    fill = jnp.arange(npd, dtype=jnp.int32)
    src_p = jnp.concatenate([edge_index[0], fill % n])
    dst_p = jnp.concatenate([edge_index[1], fill % n_pad])
    w_p = jnp.pad(edge_attr, (0, npd))
    src3 = src_p.reshape(nw, n_super, cps, chunk)
    dst3 = dst_p.reshape(nw, n_super, cps, chunk)
    w3d = w_p.reshape(nw, n_super, cps * chunk)
    batch2d = batch.reshape(n, 1)

    bemb2 = bemb.reshape(1, h)
    b1r = b1.reshape(nl, 1, h)
    b3r = b3.reshape(nl, 1, h)
    bf1r = bf1.reshape(1, 2 * h)
    wf2p = jnp.pad(Wf2, ((0, 0), (0, h - nc)))
    bf2p = jnp.pad(bf2, (0, h - nc)).reshape(1, h)

    agg_full = _make_edge_agg(n_pad, e_pad, h, n_super, cps, chunk)
    ones128 = jnp.ones((n, h), jnp.float32)

    degp = agg_full(ones128, src3, dst3, w3d)
    d0, d1 = degp[0, :n, :16], degp[1, :n, :16]

    hh, a = _tc_embed(x, Wemb, bemb2, W1[0], b1r[0], br=br)
    for l in range(nl):
        p = agg_full(a, src3, dst3, w3d)
        if l < nl - 1:
            hh, a = _tc_update(p[0, :n], p[1, :n], hh, d0, d1, W2[l], W3[l],
                               b3r[l], W1[l + 1], b1r[l + 1], br=br)
        else:
            pred = _tc_final(p[0, :n], p[1, :n], hh, d0, d1, W2[l], W3[l],
                             b3r[l], batch2d, Wf1, bf1r, wf2p, bf2p, br=br,
                             ng=ng)
    return pred[:, :nc]
